# baseline (speedup 1.0000x reference)
import functools

import jax
import jax.numpy as jnp
from jax import lax
from jax.experimental import pallas as pl
from jax.experimental.pallas import tpu as pltpu
from jax.experimental.pallas import tpu_sc as plsc

NSEG = 10000
D = 128
ROWS = 320000
NC = 2
NS = 16
NW = NC * NS
RPW = ROWS // NW
SUB = 80
CH = 2 * SUB
NFULL = 62
TAIL = RPW - NFULL * CH
NIDXROW = NFULL + 1
NSEG_PAD = 10240
SEG_PER_TILE = NSEG_PAD // NS
ZROWS = 16


def _sc_partial_segsum(src, idx4d):
    mesh = plsc.VectorSubcoreMesh(core_axis_name="c", subcore_axis_name="s")

    @functools.partial(
        pl.kernel,
        out_type=jax.ShapeDtypeStruct((NC, NSEG_PAD, D), jnp.float32),
        mesh=mesh,
        scratch_types=[
            pltpu.VMEM((CH, D), jnp.float32),
            pltpu.VMEM((CH, D), jnp.float32),
            pltpu.VMEM((2, SUB), jnp.int32),
            pltpu.VMEM((2, SUB), jnp.int32),
            pltpu.VMEM_SHARED((NSEG_PAD, D), jnp.float32),
            pltpu.SemaphoreType.DMA,
            pltpu.SemaphoreType.DMA,
            pltpu.SemaphoreType.DMA,
            pltpu.SemaphoreType.DMA,
            pltpu.SemaphoreType.DMA,
        ],
    )
    def k(src_hbm, idx_hbm, out_hbm, rows0, rows1, idxb0, idxb1, acc_sh,
          ls0, ls1, ss0, ss1, zsem):
        c = lax.axis_index("c")
        s = lax.axis_index("s")
        wid = c * NS + s
        row0 = wid * RPW

        rows = (rows0, rows1)
        idxb = (idxb0, idxb1)
        lsem = (ls0, ls1)
        ssem = (ss0, ss1)

        zeros16 = jnp.zeros((16,), jnp.float32)
        for i in range(ZROWS):
            for j in range(D // 16):
                rows0[i, pl.ds(j * 16, 16)] = zeros16
        ztpl = rows0.at[pl.ds(0, ZROWS)]
        for i in range(SEG_PER_TILE // ZROWS):
            pltpu.async_copy(
                ztpl, acc_sh.at[pl.ds(s * SEG_PER_TILE + i * ZROWS, ZROWS)],
                zsem)
        for i in range(SEG_PER_TILE // ZROWS):
            pltpu.make_async_copy(
                ztpl, acc_sh.at[pl.ds(s * SEG_PER_TILE + i * ZROWS, ZROWS)],
                zsem).wait()

        def load(g, b):
            base = pl.multiple_of(row0 + g * CH, CH)
            return (
                pltpu.make_async_copy(src_hbm.at[pl.ds(base, CH)], rows[b],
                                      lsem[b]),
                pltpu.make_async_copy(idx_hbm.at[wid, g], idxb[b], lsem[b]),
            )

        def start_load(g, b):
            for cp in load(g, b):
                cp.start()

        def wait_load(g, b):
            for cp in load(g, b):
                cp.wait()

        def scatter(b, q):
            return pltpu.make_async_copy(
                rows[b].at[pl.ds(q * SUB, SUB)],
                acc_sh.at[idxb[b].at[q]],
                ssem[b])

        start_load(0, 0)
        start_load(1, 1)
        plsc.subcore_barrier()

        def half(i, g, b):
            wait_load(g, b)
            scatter(b, 0).start(add=True)
            scatter(b, 1).start(add=True)

        def reload(i, g, b):
            scatter(b, 0).wait()
            scatter(b, 1).wait()

            @pl.when(i < NFULL // 2 - 1)
            def _():
                start_load(g, b)

        def body(i, _):
            g = 2 * i
            half(i, g, 0)
            half(i, g + 1, 1)
            reload(i, g + 2, 0)
            reload(i, g + 3, 1)
            return 0

        lax.fori_loop(0, NFULL // 2, body, 0)

        tbase = pl.multiple_of(row0 + NFULL * CH, SUB)
        tsrc = pltpu.make_async_copy(
            src_hbm.at[pl.ds(tbase, TAIL)], rows0.at[pl.ds(0, TAIL)], ls0)
        tidx = pltpu.make_async_copy(idx_hbm.at[wid, NFULL], idxb0, ls0)
        tsrc.start()
        tidx.start()
        tsrc.wait()
        tidx.wait()
        pltpu.make_async_copy(
            rows0.at[pl.ds(0, TAIL)], acc_sh.at[idxb0.at[0]], ss0
        ).start(add=True)
        pltpu.make_async_copy(
            rows0.at[pl.ds(0, TAIL)], acc_sh.at[idxb0.at[0]], ss0).wait()

        plsc.subcore_barrier()
        pltpu.sync_copy(
            acc_sh.at[pl.ds(s * SEG_PER_TILE, SEG_PER_TILE)],
            out_hbm.at[c, pl.ds(s * SEG_PER_TILE, SEG_PER_TILE)],
        )

    return k(src, idx4d)


def _tc_add_partials(partials):
    def body(p_ref, o_ref):
        o_ref[...] = p_ref[0] + p_ref[1]

    blk = NSEG // 10
    return pl.pallas_call(
        body,
        out_shape=jax.ShapeDtypeStruct((NSEG, D), jnp.float32),
        grid=(NSEG // blk,),
        in_specs=[pl.BlockSpec((NC, blk, D), lambda i: (0, i, 0))],
        out_specs=pl.BlockSpec((blk, D), lambda i: (i, 0)),
    )(partials)


def kernel(src, index, dim_size):
    idx = index.astype(jnp.int32).reshape(NW, RPW // SUB, SUB)
    idx4d = jnp.pad(idx, ((0, 0), (0, 1), (0, 0))).reshape(NW, NIDXROW, 2, SUB)
    partials = _sc_partial_segsum(src, idx4d)
    return _tc_add_partials(partials)
